# baseline (speedup 1.0000x reference)
import functools
import jax
import jax.numpy as jnp
from jax import lax
from jax.experimental import pallas as pl
from jax.experimental.pallas import tpu as pltpu
from jax.experimental.pallas import tpu_sc as plsc

D = 128
C = 128
LANES = 16


def _sc_body(x_hbm, idx_hbm, pe_hbm, out_hbm, x_v, idx_v, sem_x, sem_pe):
    nc = 2
    wid = lax.axis_index("s") * nc + lax.axis_index("c")
    n_rows = x_hbm.shape[0]
    nw = 32
    rows_per_w = n_rows // nw
    chunks = rows_per_w // C
    base = wid * rows_per_w

    def chunk_body(g, _):
        off = base + g * C
        pltpu.sync_copy(idx_hbm.at[pl.ds(off, C)], idx_v)
        cp_x = pltpu.async_copy(x_hbm.at[pl.ds(off, C)], x_v, sem_x)
        cp_x.wait()
        cp_pe = pltpu.async_copy(pe_hbm.at[idx_v], x_v, sem_pe, add=True)
        cp_pe.wait()
        pltpu.sync_copy(x_v, out_hbm.at[pl.ds(off, C)])
        return ()

    lax.fori_loop(0, chunks, chunk_body, ())


@jax.jit
def _pe_add(x2d, idx1d, pe):
    n = x2d.shape[0]
    mesh = plsc.VectorSubcoreMesh(core_axis_name="c", subcore_axis_name="s")
    f = pl.kernel(
        _sc_body,
        out_type=jax.ShapeDtypeStruct((n, D), jnp.float32),
        mesh=mesh,
        scratch_types=[
            pltpu.VMEM((C, D), jnp.float32),
            pltpu.VMEM((C,), jnp.int32),
            pltpu.SemaphoreType.DMA,
            pltpu.SemaphoreType.DMA,
        ],
    )
    return f(x2d, idx1d, pe)


def kernel(x, segment_positions, pe):
    b, s, d = x.shape
    x2d = x.reshape(b * s, d)
    idx1d = segment_positions.reshape(b * s).astype(jnp.int32)
    out = _pe_add(x2d, idx1d, pe.astype(jnp.float32))
    return out.reshape(b, s, d)


# speedup vs baseline: 1.0973x; 1.0973x over previous
import functools
import jax
import jax.numpy as jnp
from jax import lax
from jax.experimental import pallas as pl
from jax.experimental.pallas import tpu as pltpu
from jax.experimental.pallas import tpu_sc as plsc

D = 128
C = 128
NBUF = 4
NW = 32


def _sc_body(x_hbm, idx_hbm, pe_hbm, out_hbm, x_v, idx_v, sem_x, sem_pe, sem_o):
    nc = 2
    wid = lax.axis_index("s") * nc + lax.axis_index("c")
    n_rows = x_hbm.shape[0]
    rows_per_w = n_rows // NW
    chunks = rows_per_w // C
    groups = chunks // NBUF
    base = wid * rows_per_w

    pltpu.sync_copy(idx_hbm.at[pl.ds(base, rows_per_w)], idx_v)

    def group_body(o, _):
        g0 = o * NBUF
        cps = []
        for b in range(NBUF):
            off = base + (g0 + b) * C
            cps.append(pltpu.async_copy(x_hbm.at[pl.ds(off, C)], x_v.at[b], sem_x))
        for cp in cps:
            cp.wait()
        cps = []
        for b in range(NBUF):
            loff = (g0 + b) * C
            cps.append(pltpu.async_copy(
                pe_hbm.at[idx_v.at[pl.ds(loff, C)]], x_v.at[b], sem_pe, add=True))
        for cp in cps:
            cp.wait()
        cps = []
        for b in range(NBUF):
            off = base + (g0 + b) * C
            cps.append(pltpu.async_copy(x_v.at[b], out_hbm.at[pl.ds(off, C)], sem_o))
        for cp in cps:
            cp.wait()
        return ()

    lax.fori_loop(0, groups, group_body, ())


@jax.jit
def _pe_add(x2d, idx1d, pe):
    n = x2d.shape[0]
    mesh = plsc.VectorSubcoreMesh(core_axis_name="c", subcore_axis_name="s")
    f = pl.kernel(
        _sc_body,
        out_type=jax.ShapeDtypeStruct((n, D), jnp.float32),
        mesh=mesh,
        scratch_types=[
            pltpu.VMEM((NBUF, C, D), jnp.float32),
            pltpu.VMEM((n // NW,), jnp.int32),
            pltpu.SemaphoreType.DMA,
            pltpu.SemaphoreType.DMA,
            pltpu.SemaphoreType.DMA,
        ],
    )
    return f(x2d, idx1d, pe)


def kernel(x, segment_positions, pe):
    b, s, d = x.shape
    x2d = x.reshape(b * s, d)
    idx1d = segment_positions.reshape(b * s).astype(jnp.int32)
    out = _pe_add(x2d, idx1d, pe.astype(jnp.float32))
    return out.reshape(b, s, d)
